# baseline (speedup 1.0000x reference)
import functools

import jax
import jax.numpy as jnp
from jax import lax
from jax.experimental import pallas as pl
from jax.experimental.pallas import tpu as pltpu
from jax.experimental.pallas import tpu_sc as plsc

D_MODEL = 128
MAX_POS = 255
SEQ = 512
PAD = 1024

NUM_CORES = 2
NUM_SUBCORES = 16
NUM_WORKERS = NUM_CORES * NUM_SUBCORES

SC_ROWS = 192
BLK_I = 8


def _sc_part(pe, B):
    rows_per_worker = SC_ROWS // NUM_WORKERS
    mesh = plsc.VectorSubcoreMesh(core_axis_name="c", subcore_axis_name="s")

    @functools.partial(
        pl.kernel,
        out_type=jax.ShapeDtypeStruct((B, SEQ, SEQ, D_MODEL), jnp.float32),
        mesh=mesh,
        scratch_types=[
            pltpu.VMEM_SHARED((PAD, D_MODEL), jnp.float32),
            pltpu.VMEM((D_MODEL,), jnp.float32),
            pltpu.VMEM((NUM_SUBCORES, D_MODEL), jnp.float32),
        ],
    )
    def sc_kernel(pe_hbm, out_hbm, p_sh, row_v, rep_v):
        c = lax.axis_index("c")
        s = lax.axis_index("s")

        pltpu.sync_copy(
            pe_hbm.at[pl.ds(s * 32, 32)], p_sh.at[pl.ds(256 + s * 32, 32)]
        )

        @pl.when(s == NUM_SUBCORES - 1)
        def _fix_last():
            pltpu.sync_copy(pe_hbm.at[2 * MAX_POS], p_sh.at[PAD - 257])

        def replicate(src_row):
            pltpu.sync_copy(pe_hbm.at[src_row], row_v)
            for c16 in range(D_MODEL // 16):
                v = row_v[pl.ds(c16 * 16, 16)]
                for r in range(NUM_SUBCORES):
                    rep_v[r, pl.ds(c16 * 16, 16)] = v

        replicate(0)
        pltpu.sync_copy(rep_v, p_sh.at[pl.ds(s * 16, 16)])
        replicate(2 * MAX_POS)
        pltpu.sync_copy(rep_v, p_sh.at[pl.ds(768 + s * 16, 16)])

        plsc.subcore_barrier()

        wid = s * NUM_CORES + c
        i_base = wid * rows_per_worker
        for b in range(B):
            for k in range(rows_per_worker):
                i = i_base + k
                pltpu.sync_copy(
                    p_sh.at[pl.ds((SEQ - 1) - i, SEQ)], out_hbm.at[b, i]
                )

    return sc_kernel(pe)


def _tc_part(pe, partial_out):
    B = partial_out.shape[0]
    blocks_done = SC_ROWS // BLK_I

    def tc_body(pe_ref, prev_ref, out_ref, p_scr):
        b = pl.program_id(0)
        ib = pl.program_id(1)

        @pl.when((b == 0) & (ib == 0))
        def _build():
            p_scr[pl.ds(256, 512), :] = pe_ref[...]
            p_scr[pl.ds(0, 256), :] = jnp.broadcast_to(
                pe_ref[0:1, :], (256, D_MODEL)
            )
            p_scr[pl.ds(767, 257), :] = jnp.broadcast_to(
                pe_ref[510:511, :], (257, D_MODEL)
            )

        for r in range(BLK_I):
            i = SC_ROWS + ib * BLK_I + r
            out_ref[0, r] = p_scr[pl.ds((SEQ - 1) - i, SEQ), :]

    return pl.pallas_call(
        tc_body,
        grid=(B, (SEQ - SC_ROWS) // BLK_I),
        in_specs=[
            pl.BlockSpec((SEQ, D_MODEL), lambda b, ib: (0, 0)),
            pl.BlockSpec(memory_space=pl.ANY),
        ],
        out_specs=pl.BlockSpec(
            (1, BLK_I, SEQ, D_MODEL), lambda b, ib: (b, ib + blocks_done, 0, 0)
        ),
        out_shape=jax.ShapeDtypeStruct((B, SEQ, SEQ, D_MODEL), jnp.float32),
        scratch_shapes=[pltpu.VMEM((PAD, D_MODEL), jnp.float32)],
        input_output_aliases={1: 0},
    )(pe, partial_out)


def kernel(x, pe):
    B, S = x.shape
    assert S == SEQ and pe.shape == (SEQ, D_MODEL)
    partial = _sc_part(pe, B)
    return _tc_part(pe, partial)


# speedup vs baseline: 1.0028x; 1.0028x over previous
import functools

import jax
import jax.numpy as jnp
from jax import lax
from jax.experimental import pallas as pl
from jax.experimental.pallas import tpu as pltpu
from jax.experimental.pallas import tpu_sc as plsc

D_MODEL = 128
MAX_POS = 255
SEQ = 512
PAD = 1024

NUM_CORES = 2
NUM_SUBCORES = 16
NUM_WORKERS = NUM_CORES * NUM_SUBCORES

SC_ROWS = 192
BLK_I = 8


def _sc_part(pe, B):
    rows_per_worker = SC_ROWS // NUM_WORKERS
    mesh = plsc.VectorSubcoreMesh(core_axis_name="c", subcore_axis_name="s")

    @functools.partial(
        pl.kernel,
        out_type=jax.ShapeDtypeStruct((B, SEQ, SEQ, D_MODEL), jnp.float32),
        mesh=mesh,
        scratch_types=[
            pltpu.VMEM_SHARED((PAD, D_MODEL), jnp.float32),
            pltpu.VMEM((D_MODEL,), jnp.float32),
            pltpu.VMEM((NUM_SUBCORES, D_MODEL), jnp.float32),
            pltpu.SemaphoreType.DMA,
            pltpu.SemaphoreType.DMA,
        ],
    )
    def sc_kernel(pe_hbm, out_hbm, p_sh, row_v, rep_v, sem_set, sem_out):
        c = lax.axis_index("c")
        s = lax.axis_index("s")

        setup = [
            pltpu.async_copy(
                pe_hbm.at[pl.ds(s * 32, 32)],
                p_sh.at[pl.ds(256 + s * 32, 32)],
                sem_set,
            )
        ]

        def replicate(src_row):
            pltpu.sync_copy(pe_hbm.at[src_row], row_v)
            for c16 in range(D_MODEL // 16):
                v = row_v[pl.ds(c16 * 16, 16)]
                for r in range(NUM_SUBCORES):
                    rep_v[r, pl.ds(c16 * 16, 16)] = v

        replicate(0)
        setup.append(
            pltpu.async_copy(rep_v, p_sh.at[pl.ds(s * 16, 16)], sem_set)
        )
        for h in setup:
            h.wait()
        replicate(2 * MAX_POS)
        setup = [
            pltpu.async_copy(rep_v, p_sh.at[pl.ds(768 + s * 16, 16)], sem_set)
        ]
        @pl.when(s == NUM_SUBCORES - 1)
        def _fix_last():
            pltpu.sync_copy(pe_hbm.at[2 * MAX_POS], p_sh.at[PAD - 257])

        for h in setup:
            h.wait()
        plsc.subcore_barrier()

        wid = s * NUM_CORES + c
        i_base = wid * rows_per_worker
        handles = []
        for k in range(rows_per_worker):
            for b in range(B):
                i = i_base + k
                handles.append(
                    pltpu.async_copy(
                        p_sh.at[pl.ds((SEQ - 1) - i, SEQ)],
                        out_hbm.at[b, i],
                        sem_out,
                    )
                )
        for h in handles:
            h.wait()

    return sc_kernel(pe)


def _tc_part(pe, partial_out):
    B = partial_out.shape[0]
    blocks_done = SC_ROWS // BLK_I

    def tc_body(pe_ref, prev_ref, out_ref, p_scr):
        b = pl.program_id(0)
        ib = pl.program_id(1)

        @pl.when((b == 0) & (ib == 0))
        def _build():
            p_scr[pl.ds(256, 512), :] = pe_ref[...]
            p_scr[pl.ds(0, 256), :] = jnp.broadcast_to(
                pe_ref[0:1, :], (256, D_MODEL)
            )
            p_scr[pl.ds(767, 257), :] = jnp.broadcast_to(
                pe_ref[510:511, :], (257, D_MODEL)
            )

        for r in range(BLK_I):
            i = SC_ROWS + ib * BLK_I + r
            out_ref[0, r] = p_scr[pl.ds((SEQ - 1) - i, SEQ), :]

    return pl.pallas_call(
        tc_body,
        grid=(B, (SEQ - SC_ROWS) // BLK_I),
        in_specs=[
            pl.BlockSpec((SEQ, D_MODEL), lambda b, ib: (0, 0)),
            pl.BlockSpec(memory_space=pl.ANY),
        ],
        out_specs=pl.BlockSpec(
            (1, BLK_I, SEQ, D_MODEL), lambda b, ib: (b, ib + blocks_done, 0, 0)
        ),
        out_shape=jax.ShapeDtypeStruct((B, SEQ, SEQ, D_MODEL), jnp.float32),
        scratch_shapes=[pltpu.VMEM((PAD, D_MODEL), jnp.float32)],
        input_output_aliases={1: 0},
    )(pe, partial_out)


def kernel(x, pe):
    B, S = x.shape
    assert S == SEQ and pe.shape == (SEQ, D_MODEL)
    partial = _sc_part(pe, B)
    return _tc_part(pe, partial)


# speedup vs baseline: 1.0410x; 1.0382x over previous
import functools

import jax
import jax.numpy as jnp
from jax import lax
from jax.experimental import pallas as pl
from jax.experimental.pallas import tpu as pltpu
from jax.experimental.pallas import tpu_sc as plsc

D_MODEL = 128
MAX_POS = 255
SEQ = 512
PAD = 1024

NUM_CORES = 2
NUM_SUBCORES = 16
NUM_WORKERS = NUM_CORES * NUM_SUBCORES

SC_ROWS = 128
BLK_I = 8


def _sc_part(pe, B):
    rows_per_worker = SC_ROWS // NUM_WORKERS
    mesh = plsc.VectorSubcoreMesh(core_axis_name="c", subcore_axis_name="s")

    @functools.partial(
        pl.kernel,
        out_type=jax.ShapeDtypeStruct((B, SEQ, SEQ, D_MODEL), jnp.float32),
        mesh=mesh,
        scratch_types=[
            pltpu.VMEM_SHARED((PAD, D_MODEL), jnp.float32),
            pltpu.VMEM((D_MODEL,), jnp.float32),
            pltpu.VMEM((NUM_SUBCORES, D_MODEL), jnp.float32),
            pltpu.SemaphoreType.DMA,
            pltpu.SemaphoreType.DMA,
        ],
    )
    def sc_kernel(pe_hbm, out_hbm, p_sh, row_v, rep_v, sem_set, sem_out):
        c = lax.axis_index("c")
        s = lax.axis_index("s")

        setup = [
            pltpu.async_copy(
                pe_hbm.at[pl.ds(s * 32, 32)],
                p_sh.at[pl.ds(256 + s * 32, 32)],
                sem_set,
            )
        ]

        def replicate(src_row):
            pltpu.sync_copy(pe_hbm.at[src_row], row_v)
            for c16 in range(D_MODEL // 16):
                v = row_v[pl.ds(c16 * 16, 16)]
                for r in range(NUM_SUBCORES):
                    rep_v[r, pl.ds(c16 * 16, 16)] = v

        replicate(0)
        setup.append(
            pltpu.async_copy(rep_v, p_sh.at[pl.ds(s * 16, 16)], sem_set)
        )
        for h in setup:
            h.wait()
        replicate(2 * MAX_POS)
        setup = [
            pltpu.async_copy(rep_v, p_sh.at[pl.ds(768 + s * 16, 16)], sem_set)
        ]
        @pl.when(s == NUM_SUBCORES - 1)
        def _fix_last():
            pltpu.sync_copy(pe_hbm.at[2 * MAX_POS], p_sh.at[PAD - 257])

        for h in setup:
            h.wait()
        plsc.subcore_barrier()

        wid = s * NUM_CORES + c
        i_base = wid * rows_per_worker
        handles = []
        for k in range(rows_per_worker):
            for b in range(B):
                i = i_base + k
                handles.append(
                    pltpu.async_copy(
                        p_sh.at[pl.ds((SEQ - 1) - i, SEQ)],
                        out_hbm.at[b, i],
                        sem_out,
                    )
                )
        for h in handles:
            h.wait()

    return sc_kernel(pe)


def _tc_part(pe, partial_out):
    B = partial_out.shape[0]
    blocks_done = SC_ROWS // BLK_I

    def tc_body(pe_ref, prev_ref, out_ref, p_scr):
        b = pl.program_id(0)
        ib = pl.program_id(1)

        @pl.when((b == 0) & (ib == 0))
        def _build():
            p_scr[pl.ds(256, 512), :] = pe_ref[...]
            p_scr[pl.ds(0, 256), :] = jnp.broadcast_to(
                pe_ref[0:1, :], (256, D_MODEL)
            )
            p_scr[pl.ds(767, 257), :] = jnp.broadcast_to(
                pe_ref[510:511, :], (257, D_MODEL)
            )

        for r in range(BLK_I):
            i = SC_ROWS + ib * BLK_I + r
            out_ref[0, r] = p_scr[pl.ds((SEQ - 1) - i, SEQ), :]

    return pl.pallas_call(
        tc_body,
        grid=(B, (SEQ - SC_ROWS) // BLK_I),
        in_specs=[
            pl.BlockSpec((SEQ, D_MODEL), lambda b, ib: (0, 0)),
            pl.BlockSpec(memory_space=pl.ANY),
        ],
        out_specs=pl.BlockSpec(
            (1, BLK_I, SEQ, D_MODEL), lambda b, ib: (b, ib + blocks_done, 0, 0)
        ),
        out_shape=jax.ShapeDtypeStruct((B, SEQ, SEQ, D_MODEL), jnp.float32),
        scratch_shapes=[pltpu.VMEM((PAD, D_MODEL), jnp.float32)],
        input_output_aliases={1: 0},
    )(pe, partial_out)


def kernel(x, pe):
    B, S = x.shape
    assert S == SEQ and pe.shape == (SEQ, D_MODEL)
    partial = _sc_part(pe, B)
    return _tc_part(pe, partial)


# speedup vs baseline: 1.1365x; 1.0917x over previous
import functools

import jax
import jax.numpy as jnp
from jax import lax
from jax.experimental import pallas as pl
from jax.experimental.pallas import tpu as pltpu
from jax.experimental.pallas import tpu_sc as plsc

D_MODEL = 128
MAX_POS = 255
SEQ = 512
PAD = 1024

NUM_CORES = 2
NUM_SUBCORES = 16
NUM_WORKERS = NUM_CORES * NUM_SUBCORES

SC_ROWS = 128
BLK_I = 16


def _sc_part(pe, B):
    rows_per_worker = SC_ROWS // NUM_WORKERS
    mesh = plsc.VectorSubcoreMesh(core_axis_name="c", subcore_axis_name="s")

    @functools.partial(
        pl.kernel,
        out_type=jax.ShapeDtypeStruct((B, SEQ, SEQ, D_MODEL), jnp.float32),
        mesh=mesh,
        scratch_types=[
            pltpu.VMEM_SHARED((PAD, D_MODEL), jnp.float32),
            pltpu.VMEM((D_MODEL,), jnp.float32),
            pltpu.VMEM((NUM_SUBCORES, D_MODEL), jnp.float32),
            pltpu.SemaphoreType.DMA,
            pltpu.SemaphoreType.DMA,
        ],
    )
    def sc_kernel(pe_hbm, out_hbm, p_sh, row_v, rep_v, sem_set, sem_out):
        c = lax.axis_index("c")
        s = lax.axis_index("s")

        setup = [
            pltpu.async_copy(
                pe_hbm.at[pl.ds(s * 32, 32)],
                p_sh.at[pl.ds(256 + s * 32, 32)],
                sem_set,
            )
        ]

        def replicate(src_row):
            pltpu.sync_copy(pe_hbm.at[src_row], row_v)
            for c16 in range(D_MODEL // 16):
                v = row_v[pl.ds(c16 * 16, 16)]
                for r in range(NUM_SUBCORES):
                    rep_v[r, pl.ds(c16 * 16, 16)] = v

        replicate(0)
        setup.append(
            pltpu.async_copy(rep_v, p_sh.at[pl.ds(s * 16, 16)], sem_set)
        )
        for h in setup:
            h.wait()
        replicate(2 * MAX_POS)
        setup = [
            pltpu.async_copy(rep_v, p_sh.at[pl.ds(768 + s * 16, 16)], sem_set)
        ]
        @pl.when(s == NUM_SUBCORES - 1)
        def _fix_last():
            pltpu.sync_copy(pe_hbm.at[2 * MAX_POS], p_sh.at[PAD - 257])

        for h in setup:
            h.wait()
        plsc.subcore_barrier()

        wid = s * NUM_CORES + c
        i_base = wid * rows_per_worker
        handles = []
        for k in range(rows_per_worker):
            for b in range(B):
                i = i_base + k
                handles.append(
                    pltpu.async_copy(
                        p_sh.at[pl.ds((SEQ - 1) - i, SEQ)],
                        out_hbm.at[b, i],
                        sem_out,
                    )
                )
        for h in handles:
            h.wait()

    return sc_kernel(pe)


def _tc_part(pe, partial_out):
    B = partial_out.shape[0]
    blocks_done = SC_ROWS // BLK_I

    def tc_body(pe_ref, prev_ref, out_ref, p_scr):
        b = pl.program_id(0)
        ib = pl.program_id(1)

        @pl.when((b == 0) & (ib == 0))
        def _build():
            p_scr[pl.ds(256, 512), :] = pe_ref[...]
            p_scr[pl.ds(0, 256), :] = jnp.broadcast_to(
                pe_ref[0:1, :], (256, D_MODEL)
            )
            p_scr[pl.ds(767, 257), :] = jnp.broadcast_to(
                pe_ref[510:511, :], (257, D_MODEL)
            )

        for r in range(BLK_I):
            i = SC_ROWS + ib * BLK_I + r
            out_ref[0, r] = p_scr[pl.ds((SEQ - 1) - i, SEQ), :]

    return pl.pallas_call(
        tc_body,
        grid=(B, (SEQ - SC_ROWS) // BLK_I),
        in_specs=[
            pl.BlockSpec((SEQ, D_MODEL), lambda b, ib: (0, 0)),
            pl.BlockSpec(memory_space=pl.ANY),
        ],
        out_specs=pl.BlockSpec(
            (1, BLK_I, SEQ, D_MODEL), lambda b, ib: (b, ib + blocks_done, 0, 0)
        ),
        out_shape=jax.ShapeDtypeStruct((B, SEQ, SEQ, D_MODEL), jnp.float32),
        scratch_shapes=[pltpu.VMEM((PAD, D_MODEL), jnp.float32)],
        input_output_aliases={1: 0},
    )(pe, partial_out)


def kernel(x, pe):
    B, S = x.shape
    assert S == SEQ and pe.shape == (SEQ, D_MODEL)
    partial = _sc_part(pe, B)
    return _tc_part(pe, partial)
